# baseline (speedup 1.0000x reference)
import functools

import jax
import jax.numpy as jnp
from jax import lax
from jax.experimental import pallas as pl
from jax.experimental.pallas import tpu as pltpu
from jax.experimental.pallas import tpu_sc as plsc

N = 10000
E = 320000
D_IN = 128
D_HID = 64
NUM_LAYERS = 5

NCORES = 2
NSUB = 16
NTILES = NCORES * NSUB
EPT = E // NTILES
CH = 125
CHP = 128
NCH = EPT // CH
NPAD = N + 8
NBUF = 8
RPS = 632
RPS_LAST = N - 15 * RPS
DEG_W = 8


def _mesh():
    return plsc.VectorSubcoreMesh(core_axis_name="c", subcore_axis_name="s")


_SC_PARAMS = pltpu.CompilerParams(use_tc_tiling_on_sc=False)


def _rowwise(s, fn):
    @pl.when(s < NSUB - 1)
    def _():
        fn(pl.multiple_of(s * RPS, 8), RPS)

    @pl.when(s == NSUB - 1)
    def _():
        fn((NSUB - 1) * RPS, RPS_LAST)


@functools.partial(
    pl.kernel,
    mesh=_mesh(),
    out_type=jax.ShapeDtypeStruct((NCORES, N, DEG_W), jnp.float32),
    scratch_types=[
        pltpu.VMEM((NCH, CHP), jnp.int32),
        pltpu.VMEM((CHP, DEG_W), jnp.float32),
        pltpu.VMEM_SHARED((NPAD, DEG_W), jnp.float32),
        [pltpu.SemaphoreType.DMA] * NBUF,
    ],
    compiler_params=_SC_PARAMS,
)
def _deg_pass(dst_hbm, ones_hbm, zeros_hbm, out_hbm, dst_v, ones_v, deg_sh,
              dsems):
    c = lax.axis_index("c")
    s = lax.axis_index("s")
    wid = c * NSUB + s
    pltpu.sync_copy(dst_hbm.at[wid], dst_v)
    pltpu.sync_copy(ones_hbm, ones_v)
    _rowwise(s, lambda base, cnt: pltpu.sync_copy(
        zeros_hbm.at[pl.ds(base, cnt)], deg_sh.at[pl.ds(base, cnt)]))
    plsc.subcore_barrier()

    def body(k, carry):
        for t in range(NBUF):
            j = k * NBUF + t

            @pl.when(j >= NBUF)
            def _():
                pltpu.make_async_copy(
                    ones_v, deg_sh.at[dst_v.at[j - NBUF]], dsems[t]).wait()
            pltpu.async_copy(ones_v, deg_sh.at[dst_v.at[j]], dsems[t],
                             add=True)
        return carry

    lax.fori_loop(0, NCH // NBUF, body, 0)
    for j in range(NCH - NBUF, NCH):
        pltpu.make_async_copy(
            ones_v, deg_sh.at[dst_v.at[j]], dsems[j % NBUF]).wait()
    plsc.subcore_barrier()
    _rowwise(s, lambda base, cnt: pltpu.sync_copy(
        deg_sh.at[pl.ds(base, cnt)], out_hbm.at[c].at[pl.ds(base, cnt)]))


@functools.partial(
    pl.kernel,
    mesh=_mesh(),
    out_type=jax.ShapeDtypeStruct((NCORES, N, D_HID), jnp.float32),
    scratch_types=[
        pltpu.VMEM((NCH, CHP), jnp.int32),
        pltpu.VMEM((NCH, CHP), jnp.int32),
        pltpu.VMEM((NBUF, CHP, D_HID), jnp.float32),
        pltpu.VMEM_SHARED((NPAD, D_HID), jnp.float32),
        [pltpu.SemaphoreType.DMA] * NBUF,
    ],
    compiler_params=_SC_PARAMS,
)
def _edge_pass(q_hbm, src_hbm, dst_hbm, zeros_hbm, out_hbm,
               src_v, dst_v, rows_v, agg_sh, gsems):
    c = lax.axis_index("c")
    s = lax.axis_index("s")
    wid = c * NSUB + s
    pltpu.sync_copy(src_hbm.at[wid], src_v)
    pltpu.sync_copy(dst_hbm.at[wid], dst_v)
    _rowwise(s, lambda base, cnt: pltpu.sync_copy(
        zeros_hbm.at[pl.ds(base, cnt)], agg_sh.at[pl.ds(base, cnt)]))
    plsc.subcore_barrier()

    for b in range(NBUF):
        pltpu.async_copy(q_hbm.at[src_v.at[b]], rows_v.at[b], gsems[b])

    def body(k, carry):
        for t in range(NBUF):
            j = k * NBUF + t
            pltpu.make_async_copy(
                q_hbm.at[src_v.at[j]], rows_v.at[t], gsems[t]).wait()
            pltpu.sync_copy(rows_v.at[t], agg_sh.at[dst_v.at[j]], add=True)

            @pl.when(j + NBUF < NCH)
            def _():
                pltpu.async_copy(
                    q_hbm.at[src_v.at[j + NBUF]], rows_v.at[t], gsems[t])
        return carry

    lax.fori_loop(0, NCH // NBUF, body, 0)
    plsc.subcore_barrier()
    _rowwise(s, lambda base, cnt: pltpu.sync_copy(
        agg_sh.at[pl.ds(base, cnt)], out_hbm.at[c].at[pl.ds(base, cnt)]))


def _dinv(deg_ref):
    d = deg_ref[0, :, 0:1] + deg_ref[1, :, 0:1]
    return lax.rsqrt(jnp.maximum(d, 1.0))


def _tc_first_fn(deg_ref, x_ref, w_ref, q_ref):
    q_ref[...] = _dinv(deg_ref) * jnp.dot(
        x_ref[...], w_ref[...], preferred_element_type=jnp.float32)


def _tc_mid_fn(deg_ref, s_ref, b_ref, w_ref, q_ref):
    dinv = _dinv(deg_ref)
    h = jnp.maximum(dinv * (s_ref[0] + s_ref[1]) + b_ref[...], 0.0)
    q_ref[...] = dinv * jnp.dot(h, w_ref[...],
                                preferred_element_type=jnp.float32)


def _tc_last_fn(deg_ref, s_ref, b_ref, y_ref):
    dinv = _dinv(deg_ref)
    h = jnp.maximum(dinv * (s_ref[0] + s_ref[1]) + b_ref[...], 0.0)
    y_ref[...] = jnp.sum(h, axis=1, keepdims=True)


_tc_first = pl.pallas_call(
    _tc_first_fn, out_shape=jax.ShapeDtypeStruct((N, D_HID), jnp.float32))
_tc_mid = pl.pallas_call(
    _tc_mid_fn, out_shape=jax.ShapeDtypeStruct((N, D_HID), jnp.float32))
_tc_last = pl.pallas_call(
    _tc_last_fn, out_shape=jax.ShapeDtypeStruct((N, 1), jnp.float32))


def kernel(x, edge_index, W_in, b_in, W_hid, b_hid, W_out, b_out):
    pad = ((0, 0), (0, 0), (0, CHP - CH))
    src = jnp.pad(edge_index[0].reshape(NTILES, NCH, CH), pad)
    dst = jnp.pad(edge_index[1].reshape(NTILES, NCH, CH), pad,
                  constant_values=N)
    zeros_d = jnp.zeros((N, D_HID), jnp.float32)
    zeros_g = jnp.zeros((N, DEG_W), jnp.float32)
    ones_g = jnp.ones((CHP, DEG_W), jnp.float32)

    deg = _deg_pass(dst, ones_g, zeros_g)
    q = _tc_first(deg, x[0], W_in)

    biases = [b_in, b_hid[0], b_hid[1], b_hid[2]]
    weights = [W_hid[0], W_hid[1], W_hid[2], W_out]
    for b, w in zip(biases, weights):
        s = _edge_pass(q, src, dst, zeros_d)
        q = _tc_mid(deg, s, b.reshape(1, D_HID), w)

    s = _edge_pass(q, src, dst, zeros_d)
    y = _tc_last(deg, s, b_out.reshape(1, D_HID))
    return y[None]


# speedup vs baseline: 1.0005x; 1.0005x over previous
import functools

import jax
import jax.numpy as jnp
from jax import lax
from jax.experimental import pallas as pl
from jax.experimental.pallas import tpu as pltpu
from jax.experimental.pallas import tpu_sc as plsc

N = 10000
E = 320000
D_IN = 128
D_HID = 64
NUM_LAYERS = 5

NCORES = 2
NSUB = 16
NTILES = NCORES * NSUB
EPT = E // NTILES
CH = 125
CHP = 128
NCH = EPT // CH
NPAD = N + NSUB
NBUF = 8
RPS = 632
RPS_LAST = N - 15 * RPS
DEG_W = 8


def _mesh():
    return plsc.VectorSubcoreMesh(core_axis_name="c", subcore_axis_name="s")


_SC_PARAMS = pltpu.CompilerParams(use_tc_tiling_on_sc=False)


def _rowwise(s, fn):
    @pl.when(s < NSUB - 1)
    def _():
        fn(pl.multiple_of(s * RPS, 8), RPS)

    @pl.when(s == NSUB - 1)
    def _():
        fn((NSUB - 1) * RPS, RPS_LAST)


@functools.partial(
    pl.kernel,
    mesh=_mesh(),
    out_type=jax.ShapeDtypeStruct((NCORES, N, DEG_W), jnp.float32),
    scratch_types=[
        pltpu.VMEM((NCH, CHP), jnp.int32),
        pltpu.VMEM((CHP, DEG_W), jnp.float32),
        pltpu.VMEM_SHARED((NPAD, DEG_W), jnp.float32),
        [pltpu.SemaphoreType.DMA] * NBUF,
    ],
    compiler_params=_SC_PARAMS,
)
def _deg_pass(dst_hbm, ones_hbm, zeros_hbm, out_hbm, dst_v, ones_v, deg_sh,
              dsems):
    c = lax.axis_index("c")
    s = lax.axis_index("s")
    wid = c * NSUB + s
    pltpu.sync_copy(dst_hbm.at[wid], dst_v)
    pltpu.sync_copy(ones_hbm, ones_v)
    _rowwise(s, lambda base, cnt: pltpu.sync_copy(
        zeros_hbm.at[pl.ds(base, cnt)], deg_sh.at[pl.ds(base, cnt)]))
    plsc.subcore_barrier()

    def body(k, carry):
        for t in range(NBUF):
            j = k * NBUF + t

            @pl.when(j >= NBUF)
            def _():
                pltpu.make_async_copy(
                    ones_v, deg_sh.at[dst_v.at[j - NBUF]], dsems[t]).wait()
            pltpu.async_copy(ones_v, deg_sh.at[dst_v.at[j]], dsems[t],
                             add=True)
        return carry

    lax.fori_loop(0, NCH // NBUF, body, 0)
    for j in range(NCH - NBUF, NCH):
        pltpu.make_async_copy(
            ones_v, deg_sh.at[dst_v.at[j]], dsems[j % NBUF]).wait()
    plsc.subcore_barrier()
    _rowwise(s, lambda base, cnt: pltpu.sync_copy(
        deg_sh.at[pl.ds(base, cnt)], out_hbm.at[c].at[pl.ds(base, cnt)]))


@functools.partial(
    pl.kernel,
    mesh=_mesh(),
    out_type=jax.ShapeDtypeStruct((NCORES, N, D_HID), jnp.float32),
    scratch_types=[
        pltpu.VMEM((NCH, CHP), jnp.int32),
        pltpu.VMEM((NCH, CHP), jnp.int32),
        pltpu.VMEM((NBUF, CHP, D_HID), jnp.float32),
        pltpu.VMEM_SHARED((NPAD, D_HID), jnp.float32),
        [pltpu.SemaphoreType.DMA] * NBUF,
    ],
    compiler_params=_SC_PARAMS,
)
def _edge_pass(q_hbm, src_hbm, dst_hbm, zeros_hbm, out_hbm,
               src_v, dst_v, rows_v, agg_sh, gsems):
    c = lax.axis_index("c")
    s = lax.axis_index("s")
    wid = c * NSUB + s
    pltpu.sync_copy(src_hbm.at[wid], src_v)
    pltpu.sync_copy(dst_hbm.at[wid], dst_v)
    _rowwise(s, lambda base, cnt: pltpu.sync_copy(
        zeros_hbm.at[pl.ds(base, cnt)], agg_sh.at[pl.ds(base, cnt)]))
    plsc.subcore_barrier()

    for b in range(NBUF):
        pltpu.async_copy(q_hbm.at[src_v.at[b]], rows_v.at[b], gsems[b])

    def body(k, carry):
        for t in range(NBUF):
            j = k * NBUF + t
            pltpu.make_async_copy(
                q_hbm.at[src_v.at[j]], rows_v.at[t], gsems[t]).wait()
            pltpu.sync_copy(rows_v.at[t], agg_sh.at[dst_v.at[j]], add=True)

            @pl.when(j + NBUF < NCH)
            def _():
                pltpu.async_copy(
                    q_hbm.at[src_v.at[j + NBUF]], rows_v.at[t], gsems[t])
        return carry

    lax.fori_loop(0, NCH // NBUF, body, 0)
    plsc.subcore_barrier()
    _rowwise(s, lambda base, cnt: pltpu.sync_copy(
        agg_sh.at[pl.ds(base, cnt)], out_hbm.at[c].at[pl.ds(base, cnt)]))


def _dinv(deg_ref):
    d = deg_ref[0, :, 0:1] + deg_ref[1, :, 0:1]
    return lax.rsqrt(jnp.maximum(d, 1.0))


def _tc_first_fn(deg_ref, x_ref, w_ref, q_ref):
    q_ref[...] = _dinv(deg_ref) * jnp.dot(
        x_ref[...], w_ref[...], preferred_element_type=jnp.float32)


def _tc_mid_fn(deg_ref, s_ref, b_ref, w_ref, q_ref):
    dinv = _dinv(deg_ref)
    h = jnp.maximum(dinv * (s_ref[0] + s_ref[1]) + b_ref[...], 0.0)
    q_ref[...] = dinv * jnp.dot(h, w_ref[...],
                                preferred_element_type=jnp.float32)


def _tc_last_fn(deg_ref, s_ref, b_ref, y_ref):
    dinv = _dinv(deg_ref)
    h = jnp.maximum(dinv * (s_ref[0] + s_ref[1]) + b_ref[...], 0.0)
    y_ref[...] = jnp.sum(h, axis=1, keepdims=True)


_tc_first = pl.pallas_call(
    _tc_first_fn, out_shape=jax.ShapeDtypeStruct((N, D_HID), jnp.float32))
_tc_mid = pl.pallas_call(
    _tc_mid_fn, out_shape=jax.ShapeDtypeStruct((N, D_HID), jnp.float32))
_tc_last = pl.pallas_call(
    _tc_last_fn, out_shape=jax.ShapeDtypeStruct((N, 1), jnp.float32))


def kernel(x, edge_index, W_in, b_in, W_hid, b_hid, W_out, b_out):
    pad = ((0, 0), (0, 0), (0, CHP - CH))
    src = jnp.pad(edge_index[0].reshape(NTILES, NCH, CH), pad)
    sac = (N + jnp.arange(NTILES, dtype=jnp.int32) % NSUB)[:, None, None]
    dst3 = edge_index[1].reshape(NTILES, NCH, CH)
    dst = jnp.concatenate(
        [dst3, jnp.broadcast_to(sac, (NTILES, NCH, CHP - CH))], axis=2)
    zeros_d = jnp.zeros((N, D_HID), jnp.float32)
    zeros_g = jnp.zeros((N, DEG_W), jnp.float32)
    ones_g = jnp.ones((CHP, DEG_W), jnp.float32)

    deg = _deg_pass(dst, ones_g, zeros_g)
    q = _tc_first(deg, x[0], W_in)

    biases = [b_in, b_hid[0], b_hid[1], b_hid[2]]
    weights = [W_hid[0], W_hid[1], W_hid[2], W_out]
    for b, w in zip(biases, weights):
        s = _edge_pass(q, src, dst, zeros_d)
        q = _tc_mid(deg, s, b.reshape(1, D_HID), w)

    s = _edge_pass(q, src, dst, zeros_d)
    y = _tc_last(deg, s, b_out.reshape(1, D_HID))
    return y[None]


# speedup vs baseline: 2.6235x; 2.6222x over previous
import functools

import jax
import jax.numpy as jnp
from jax import lax
from jax.experimental import pallas as pl
from jax.experimental.pallas import tpu as pltpu
from jax.experimental.pallas import tpu_sc as plsc

N = 10000
E = 320000
D_IN = 128
D_HID = 64
NUM_LAYERS = 5

NCORES = 2
NSUB = 16
NTILES = NCORES * NSUB
EPT = E // NTILES
CH = 125
CHP = 125
NCH = EPT // CH
NPAD = N + NSUB
NBUF = 8
RPS = 632
RPS_LAST = N - 15 * RPS
DEG_W = 8


def _mesh():
    return plsc.VectorSubcoreMesh(core_axis_name="c", subcore_axis_name="s")


_SC_PARAMS = pltpu.CompilerParams(use_tc_tiling_on_sc=False)


def _rowwise(s, fn):
    @pl.when(s < NSUB - 1)
    def _():
        fn(pl.multiple_of(s * RPS, 8), RPS)

    @pl.when(s == NSUB - 1)
    def _():
        fn((NSUB - 1) * RPS, RPS_LAST)


@functools.partial(
    pl.kernel,
    mesh=_mesh(),
    out_type=jax.ShapeDtypeStruct((NCORES, N, DEG_W), jnp.float32),
    scratch_types=[
        pltpu.VMEM((NCH, CHP), jnp.int32),
        pltpu.VMEM((CHP, DEG_W), jnp.float32),
        pltpu.VMEM_SHARED((NPAD, DEG_W), jnp.float32),
        [pltpu.SemaphoreType.DMA] * NBUF,
    ],
    compiler_params=_SC_PARAMS,
)
def _deg_pass(dst_hbm, ones_hbm, zeros_hbm, out_hbm, dst_v, ones_v, deg_sh,
              dsems):
    c = lax.axis_index("c")
    s = lax.axis_index("s")
    wid = c * NSUB + s
    pltpu.sync_copy(dst_hbm.at[wid], dst_v)
    pltpu.sync_copy(ones_hbm, ones_v)
    _rowwise(s, lambda base, cnt: pltpu.sync_copy(
        zeros_hbm.at[pl.ds(base, cnt)], deg_sh.at[pl.ds(base, cnt)]))
    plsc.subcore_barrier()

    def body(k, carry):
        for t in range(NBUF):
            j = k * NBUF + t

            @pl.when(j >= NBUF)
            def _():
                pltpu.make_async_copy(
                    ones_v, deg_sh.at[dst_v.at[j - NBUF]], dsems[t]).wait()
            pltpu.async_copy(ones_v, deg_sh.at[dst_v.at[j]], dsems[t],
                             add=True)
        return carry

    lax.fori_loop(0, NCH // NBUF, body, 0)
    for j in range(NCH - NBUF, NCH):
        pltpu.make_async_copy(
            ones_v, deg_sh.at[dst_v.at[j]], dsems[j % NBUF]).wait()
    plsc.subcore_barrier()
    _rowwise(s, lambda base, cnt: pltpu.sync_copy(
        deg_sh.at[pl.ds(base, cnt)], out_hbm.at[c].at[pl.ds(base, cnt)]))


@functools.partial(
    pl.kernel,
    mesh=_mesh(),
    out_type=jax.ShapeDtypeStruct((NCORES, N, D_HID), jnp.float32),
    scratch_types=[
        pltpu.VMEM((NCH, CHP), jnp.int32),
        pltpu.VMEM((NCH, CHP), jnp.int32),
        pltpu.VMEM((NBUF, CHP, D_HID), jnp.float32),
        pltpu.VMEM_SHARED((NPAD, D_HID), jnp.float32),
        [pltpu.SemaphoreType.DMA] * NBUF,
    ],
    compiler_params=_SC_PARAMS,
)
def _edge_pass(q_hbm, src_hbm, dst_hbm, zeros_hbm, out_hbm,
               src_v, dst_v, rows_v, agg_sh, gsems):
    c = lax.axis_index("c")
    s = lax.axis_index("s")
    wid = c * NSUB + s
    pltpu.sync_copy(src_hbm.at[wid], src_v)
    pltpu.sync_copy(dst_hbm.at[wid], dst_v)
    _rowwise(s, lambda base, cnt: pltpu.sync_copy(
        zeros_hbm.at[pl.ds(base, cnt)], agg_sh.at[pl.ds(base, cnt)]))
    plsc.subcore_barrier()

    for b in range(NBUF):
        pltpu.async_copy(q_hbm.at[src_v.at[b]], rows_v.at[b], gsems[b])

    def body(k, carry):
        for t in range(NBUF):
            j = k * NBUF + t
            pltpu.make_async_copy(
                q_hbm.at[src_v.at[j]], rows_v.at[t], gsems[t]).wait()
            pltpu.sync_copy(rows_v.at[t], agg_sh.at[dst_v.at[j]], add=True)

            @pl.when(j + NBUF < NCH)
            def _():
                pltpu.async_copy(
                    q_hbm.at[src_v.at[j + NBUF]], rows_v.at[t], gsems[t])
        return carry

    lax.fori_loop(0, NCH // NBUF, body, 0)
    plsc.subcore_barrier()
    _rowwise(s, lambda base, cnt: pltpu.sync_copy(
        agg_sh.at[pl.ds(base, cnt)], out_hbm.at[c].at[pl.ds(base, cnt)]))


def _dinv(deg_ref):
    d = deg_ref[0, :, 0:1] + deg_ref[1, :, 0:1]
    return lax.rsqrt(jnp.maximum(d, 1.0))


def _tc_first_fn(deg_ref, x_ref, w_ref, q_ref):
    q_ref[...] = _dinv(deg_ref) * jnp.dot(
        x_ref[...], w_ref[...], preferred_element_type=jnp.float32)


def _tc_mid_fn(deg_ref, s_ref, b_ref, w_ref, q_ref):
    dinv = _dinv(deg_ref)
    h = jnp.maximum(dinv * (s_ref[0] + s_ref[1]) + b_ref[...], 0.0)
    q_ref[...] = dinv * jnp.dot(h, w_ref[...],
                                preferred_element_type=jnp.float32)


def _tc_last_fn(deg_ref, s_ref, b_ref, y_ref):
    dinv = _dinv(deg_ref)
    h = jnp.maximum(dinv * (s_ref[0] + s_ref[1]) + b_ref[...], 0.0)
    y_ref[...] = jnp.sum(h, axis=1, keepdims=True)


_tc_first = pl.pallas_call(
    _tc_first_fn, out_shape=jax.ShapeDtypeStruct((N, D_HID), jnp.float32))
_tc_mid = pl.pallas_call(
    _tc_mid_fn, out_shape=jax.ShapeDtypeStruct((N, D_HID), jnp.float32))
_tc_last = pl.pallas_call(
    _tc_last_fn, out_shape=jax.ShapeDtypeStruct((N, 1), jnp.float32))


def kernel(x, edge_index, W_in, b_in, W_hid, b_hid, W_out, b_out):
    if CHP == CH:
        src = edge_index[0].reshape(NTILES, NCH, CH)
        dst = edge_index[1].reshape(NTILES, NCH, CH)
    else:
        pad = ((0, 0), (0, 0), (0, CHP - CH))
        src = jnp.pad(edge_index[0].reshape(NTILES, NCH, CH), pad)
        sac = (N + jnp.arange(NTILES, dtype=jnp.int32) % NSUB)[:, None, None]
        dst3 = edge_index[1].reshape(NTILES, NCH, CH)
        dst = jnp.concatenate(
            [dst3, jnp.broadcast_to(sac, (NTILES, NCH, CHP - CH))], axis=2)
    zeros_d = jnp.zeros((N, D_HID), jnp.float32)
    zeros_g = jnp.zeros((N, DEG_W), jnp.float32)
    ones_g = jnp.ones((CHP, DEG_W), jnp.float32)

    deg = _deg_pass(dst, ones_g, zeros_g)
    q = _tc_first(deg, x[0], W_in)

    biases = [b_in, b_hid[0], b_hid[1], b_hid[2]]
    weights = [W_hid[0], W_hid[1], W_hid[2], W_out]
    for b, w in zip(biases, weights):
        s = _edge_pass(q, src, dst, zeros_d)
        q = _tc_mid(deg, s, b.reshape(1, D_HID), w)

    s = _edge_pass(q, src, dst, zeros_d)
    y = _tc_last(deg, s, b_out.reshape(1, D_HID))
    return y[None]
